# baseline (speedup 1.0000x reference)
import functools

import jax
import jax.numpy as jnp
from jax import lax
from jax.experimental import pallas as pl
from jax.experimental.pallas import tpu as pltpu
from jax.experimental.pallas import tpu_sc as plsc

NC = 2
NS = 16
NW = NC * NS
CHUNK = 128
NBUF = 4


def _tc1_body(x_ref, wi_ref, bi_ref, wc_ref, h_ref):
    a = jnp.dot(x_ref[...], wi_ref[...], preferred_element_type=jnp.float32)
    a = jnp.maximum(a + bi_ref[...], 0.0)
    h_ref[...] = jnp.dot(a, wc_ref[...], preferred_element_type=jnp.float32)


def _tc2_body(degp_ref, h_ref, g_ref):
    deg = degp_ref[:, 0] + degp_ref[:, 1] + 1.0
    dinv = lax.rsqrt(deg)
    g_ref[...] = h_ref[...] * dinv[:, None]


def _tc3_body(tmp_ref, g_ref, degp_ref, wh_ref, bh_ref, wo_ref, bo_ref, y_ref):
    deg = degp_ref[:, 0] + degp_ref[:, 1] + 1.0
    dinv = lax.rsqrt(deg)
    s = (tmp_ref[0] + tmp_ref[1] + g_ref[...]) * dinv[:, None]
    z = jnp.dot(s, wh_ref[...], preferred_element_type=jnp.float32)
    z = jnp.maximum(z + bh_ref[...], 0.0)
    y_ref[...] = jnp.dot(z, wo_ref[...], preferred_element_type=jnp.float32) + bo_ref[...]


def _make_deg_kernel(n_chunks, n_pad, rows_pt):
    mesh = plsc.VectorSubcoreMesh(core_axis_name="c", subcore_axis_name="s")

    @functools.partial(
        pl.kernel,
        mesh=mesh,
        out_type=jax.ShapeDtypeStruct((NC * n_pad,), jnp.float32),
        scratch_types=[
            pltpu.VMEM((n_chunks, CHUNK), jnp.int32),
            pltpu.VMEM((CHUNK,), jnp.float32),
            pltpu.VMEM_SHARED((n_pad,), jnp.float32),
            pltpu.SemaphoreType.DMA,
        ],
        compiler_params=pltpu.CompilerParams(use_tc_tiling_on_sc=False),
    )
    def deg_kernel(row_hbm, ones_hbm, z1_hbm, deg_out, idx_v, ones_v, acc, sem):
        cid = lax.axis_index("c")
        sid = lax.axis_index("s")
        wid = cid * NS + sid
        pltpu.sync_copy(row_hbm.at[wid], idx_v)
        pltpu.sync_copy(ones_hbm, ones_v)
        pltpu.sync_copy(z1_hbm, acc.at[pl.ds(sid * rows_pt, rows_pt)])
        plsc.subcore_barrier()

        def body(j, carry):
            pltpu.sync_copy(ones_v, acc.at[idx_v.at[j]], add=True)
            return carry

        lax.fori_loop(0, n_chunks, body, 0)
        plsc.subcore_barrier()
        pltpu.sync_copy(acc.at[pl.ds(sid * rows_pt, rows_pt)],
                        deg_out.at[pl.ds(cid * n_pad + sid * rows_pt, rows_pt)])

    return deg_kernel


def _make_scatter_kernel(n_chunks, n_pad, rows_pt, h_dim):
    mesh = plsc.VectorSubcoreMesh(core_axis_name="c", subcore_axis_name="s")

    @functools.partial(
        pl.kernel,
        mesh=mesh,
        out_type=jax.ShapeDtypeStruct((NC * n_pad, h_dim), jnp.float32),
        scratch_types=[
            pltpu.VMEM((n_chunks, CHUNK), jnp.int32),
            pltpu.VMEM((n_chunks, CHUNK), jnp.int32),
            pltpu.VMEM((NBUF, CHUNK, h_dim), jnp.float32),
            pltpu.VMEM_SHARED((n_pad, h_dim), jnp.float32),
        ] + [pltpu.SemaphoreType.DMA] * NBUF,
        compiler_params=pltpu.CompilerParams(use_tc_tiling_on_sc=False),
    )
    def scatter_kernel(row_hbm, col_hbm, g_hbm, z2_hbm, tmp_out,
                       row_v, col_v, vbufs, acc, *sems):
        cid = lax.axis_index("c")
        sid = lax.axis_index("s")
        wid = cid * NS + sid
        pltpu.sync_copy(row_hbm.at[wid], row_v)
        pltpu.sync_copy(col_hbm.at[wid], col_v)
        pltpu.sync_copy(z2_hbm, acc.at[pl.ds(sid * rows_pt, rows_pt)])
        plsc.subcore_barrier()

        for b in range(NBUF):
            pltpu.async_copy(g_hbm.at[row_v.at[b]], vbufs.at[b], sems[b])

        n_groups = n_chunks // NBUF

        def group(gi, carry):
            for b in range(NBUF):
                j = gi * NBUF + b
                pltpu.make_async_copy(
                    g_hbm.at[row_v.at[j]], vbufs.at[b], sems[b]).wait()
                pltpu.sync_copy(vbufs.at[b], acc.at[col_v.at[j]], add=True)

                @pl.when(gi < n_groups - 1)
                def _():
                    pltpu.async_copy(
                        g_hbm.at[row_v.at[j + NBUF]], vbufs.at[b], sems[b])
            return carry

        lax.fori_loop(0, n_groups, group, 0)
        plsc.subcore_barrier()
        pltpu.sync_copy(acc.at[pl.ds(sid * rows_pt, rows_pt)],
                        tmp_out.at[pl.ds(cid * n_pad + sid * rows_pt, rows_pt)])

    return scatter_kernel


def kernel(x, edge_index, W_i, b_i, W_conv, W_h, b_h, W_o, b_o):
    N, D = x.shape
    H = W_i.shape[1]
    O = W_o.shape[1]
    E = edge_index.shape[1]

    BM = 1000
    assert N % BM == 0
    grid_n = N // BM

    epw = -(-E // NW)
    n_chunks = -(-(-(-epw // CHUNK)) // NBUF) * NBUF
    e_pad = n_chunks * CHUNK * NW
    rows_pt = -(-N // (NS * 8)) * 8
    n_pad = rows_pt * NS
    dump = n_pad - 1

    row = edge_index[0]
    col = edge_index[1]
    pad = e_pad - E
    dump_pad = jnp.full((pad,), dump, jnp.int32)
    row_d = jnp.concatenate([row, dump_pad]).reshape(NW, n_chunks, CHUNK)
    row_g = jnp.concatenate([row, jnp.zeros((pad,), jnp.int32)]).reshape(NW, n_chunks, CHUNK)
    col_s = jnp.concatenate([col, dump_pad]).reshape(NW, n_chunks, CHUNK)

    ones_c = jnp.ones((CHUNK,), jnp.float32)
    z1_c = jnp.zeros((rows_pt,), jnp.float32)
    z2_c = jnp.zeros((rows_pt, H), jnp.float32)

    h = pl.pallas_call(
        _tc1_body,
        grid=(grid_n,),
        in_specs=[
            pl.BlockSpec((BM, D), lambda i: (i, 0)),
            pl.BlockSpec((D, H), lambda i: (0, 0)),
            pl.BlockSpec((1, H), lambda i: (0, 0)),
            pl.BlockSpec((H, H), lambda i: (0, 0)),
        ],
        out_specs=pl.BlockSpec((BM, H), lambda i: (i, 0)),
        out_shape=jax.ShapeDtypeStruct((N, H), jnp.float32),
    )(x, W_i, b_i.reshape(1, H), W_conv)

    degp = _make_deg_kernel(n_chunks, n_pad, rows_pt)(row_d, ones_c, z1_c)
    degp = degp.reshape(NC, n_pad)[:, :N].T

    g = pl.pallas_call(
        _tc2_body,
        grid=(grid_n,),
        in_specs=[
            pl.BlockSpec((BM, NC), lambda i: (i, 0)),
            pl.BlockSpec((BM, H), lambda i: (i, 0)),
        ],
        out_specs=pl.BlockSpec((BM, H), lambda i: (i, 0)),
        out_shape=jax.ShapeDtypeStruct((N, H), jnp.float32),
    )(degp, h)

    tmpp = _make_scatter_kernel(n_chunks, n_pad, rows_pt, H)(row_g, col_s, g, z2_c)
    tmpp = tmpp.reshape(NC, n_pad, H)[:, :N]

    y = pl.pallas_call(
        _tc3_body,
        grid=(grid_n,),
        in_specs=[
            pl.BlockSpec((NC, BM, H), lambda i: (0, i, 0)),
            pl.BlockSpec((BM, H), lambda i: (i, 0)),
            pl.BlockSpec((BM, NC), lambda i: (i, 0)),
            pl.BlockSpec((H, H), lambda i: (0, 0)),
            pl.BlockSpec((1, H), lambda i: (0, 0)),
            pl.BlockSpec((H, O), lambda i: (0, 0)),
            pl.BlockSpec((1, O), lambda i: (0, 0)),
        ],
        out_specs=pl.BlockSpec((BM, O), lambda i: (i, 0)),
        out_shape=jax.ShapeDtypeStruct((N, O), jnp.float32),
    )(tmpp, g, degp, W_h, b_h.reshape(1, H), W_o, b_o.reshape(1, O))
    return y


# speedup vs baseline: 2.3621x; 2.3621x over previous
import functools

import jax
import jax.numpy as jnp
from jax import lax
from jax.experimental import pallas as pl
from jax.experimental.pallas import tpu as pltpu
from jax.experimental.pallas import tpu_sc as plsc

NC = 2
NS = 16
NW = NC * NS
CHUNK = 128
NBUF = 4


def _tc1_body(x_ref, wi_ref, bi_ref, wc_ref, h_ref):
    a = jnp.dot(x_ref[...], wi_ref[...], preferred_element_type=jnp.float32)
    a = jnp.maximum(a + bi_ref[...], 0.0)
    h_ref[...] = jnp.dot(a, wc_ref[...], preferred_element_type=jnp.float32)


def _tc2_body(degp_ref, h_ref, g_ref):
    deg = degp_ref[:, 0] + degp_ref[:, 1] + 1.0
    dinv = lax.rsqrt(deg)
    g_ref[...] = h_ref[...] * dinv[:, None]


def _tc3_body(tmp_ref, g_ref, degp_ref, wh_ref, bh_ref, wo_ref, bo_ref, y_ref):
    deg = degp_ref[:, 0] + degp_ref[:, 1] + 1.0
    dinv = lax.rsqrt(deg)
    s = (tmp_ref[0] + tmp_ref[1] + g_ref[...]) * dinv[:, None]
    z = jnp.dot(s, wh_ref[...], preferred_element_type=jnp.float32)
    z = jnp.maximum(z + bh_ref[...], 0.0)
    y_ref[...] = jnp.dot(z, wo_ref[...], preferred_element_type=jnp.float32) + bo_ref[...]


def _make_deg_kernel(n_chunks, n_pad, rows_pt):
    mesh = plsc.VectorSubcoreMesh(core_axis_name="c", subcore_axis_name="s")

    @functools.partial(
        pl.kernel,
        mesh=mesh,
        out_type=jax.ShapeDtypeStruct((NC * n_pad,), jnp.float32),
        scratch_types=[
            pltpu.VMEM((n_chunks, CHUNK), jnp.int32),
            pltpu.VMEM((CHUNK,), jnp.float32),
            pltpu.VMEM_SHARED((n_pad,), jnp.float32),
            pltpu.SemaphoreType.DMA,
        ],
        compiler_params=pltpu.CompilerParams(use_tc_tiling_on_sc=False),
    )
    def deg_kernel(row_hbm, ones_hbm, z1_hbm, deg_out, idx_v, ones_v, acc, sem):
        cid = lax.axis_index("c")
        sid = lax.axis_index("s")
        wid = cid * NS + sid
        pltpu.sync_copy(row_hbm.at[wid], idx_v)
        pltpu.sync_copy(ones_hbm, ones_v)
        pltpu.sync_copy(z1_hbm, acc.at[pl.ds(sid * rows_pt, rows_pt)])
        plsc.subcore_barrier()

        def body(j, carry):
            pltpu.sync_copy(ones_v, acc.at[idx_v.at[j]], add=True)
            return carry

        lax.fori_loop(0, n_chunks, body, 0)
        plsc.subcore_barrier()
        pltpu.sync_copy(acc.at[pl.ds(sid * rows_pt, rows_pt)],
                        deg_out.at[pl.ds(cid * n_pad + sid * rows_pt, rows_pt)])

    return deg_kernel


def _make_scatter_kernel(n_chunks, n_pad, rows_pt, h_dim):
    mesh = plsc.VectorSubcoreMesh(core_axis_name="c", subcore_axis_name="s")

    @functools.partial(
        pl.kernel,
        mesh=mesh,
        out_type=jax.ShapeDtypeStruct((NC * n_pad, h_dim), jnp.float32),
        scratch_types=[
            pltpu.VMEM((n_chunks, CHUNK), jnp.int32),
            pltpu.VMEM((n_chunks, CHUNK), jnp.int32),
            pltpu.VMEM((NBUF, CHUNK, h_dim), jnp.float32),
            pltpu.VMEM_SHARED((n_pad, h_dim), jnp.float32),
        ] + [pltpu.SemaphoreType.DMA] * NBUF,
        compiler_params=pltpu.CompilerParams(use_tc_tiling_on_sc=False),
    )
    def scatter_kernel(row_hbm, col_hbm, g_hbm, z2_hbm, tmp_out,
                       row_v, col_v, vbufs, acc, *sems):
        cid = lax.axis_index("c")
        sid = lax.axis_index("s")
        wid = cid * NS + sid
        pltpu.sync_copy(row_hbm.at[wid], row_v)
        pltpu.sync_copy(col_hbm.at[wid], col_v)
        pltpu.sync_copy(z2_hbm, acc.at[pl.ds(sid * rows_pt, rows_pt)])
        plsc.subcore_barrier()

        for b in range(NBUF):
            pltpu.async_copy(g_hbm.at[row_v.at[b]], vbufs.at[b], sems[b])

        n_groups = n_chunks // NBUF

        def group(gi, carry):
            for b in range(NBUF):
                j = gi * NBUF + b
                pltpu.make_async_copy(
                    g_hbm.at[row_v.at[j]], vbufs.at[b], sems[b]).wait()
                pltpu.sync_copy(vbufs.at[b], acc.at[col_v.at[j]], add=True)

                @pl.when(gi < n_groups - 1)
                def _():
                    pltpu.async_copy(
                        g_hbm.at[row_v.at[j + NBUF]], vbufs.at[b], sems[b])
            return carry

        lax.fori_loop(0, n_groups, group, 0)
        plsc.subcore_barrier()
        pltpu.sync_copy(acc.at[pl.ds(sid * rows_pt, rows_pt)],
                        tmp_out.at[pl.ds(cid * n_pad + sid * rows_pt, rows_pt)])

    return scatter_kernel


def kernel(x, edge_index, W_i, b_i, W_conv, W_h, b_h, W_o, b_o):
    N, D = x.shape
    H = W_i.shape[1]
    O = W_o.shape[1]
    E = edge_index.shape[1]

    BM = 1000
    assert N % BM == 0
    grid_n = N // BM

    epw = -(-E // NW)
    n_chunks = -(-(-(-epw // CHUNK)) // NBUF) * NBUF
    e_pad = n_chunks * CHUNK * NW
    rows_pt = -(-N // (NS * 8)) * 8
    n_pad = rows_pt * NS

    row = edge_index[0]
    col = edge_index[1]
    pad = e_pad - E
    pad_ar = jnp.arange(pad, dtype=jnp.int32)
    dump_pad = N + pad_ar % (n_pad - N)
    row_d = jnp.concatenate([row, dump_pad]).reshape(NW, n_chunks, CHUNK)
    row_g = jnp.concatenate([row, pad_ar % N]).reshape(NW, n_chunks, CHUNK)
    col_s = jnp.concatenate([col, dump_pad]).reshape(NW, n_chunks, CHUNK)

    ones_c = jnp.ones((CHUNK,), jnp.float32)
    z1_c = jnp.zeros((rows_pt,), jnp.float32)
    z2_c = jnp.zeros((rows_pt, H), jnp.float32)

    h = pl.pallas_call(
        _tc1_body,
        grid=(grid_n,),
        in_specs=[
            pl.BlockSpec((BM, D), lambda i: (i, 0)),
            pl.BlockSpec((D, H), lambda i: (0, 0)),
            pl.BlockSpec((1, H), lambda i: (0, 0)),
            pl.BlockSpec((H, H), lambda i: (0, 0)),
        ],
        out_specs=pl.BlockSpec((BM, H), lambda i: (i, 0)),
        out_shape=jax.ShapeDtypeStruct((N, H), jnp.float32),
    )(x, W_i, b_i.reshape(1, H), W_conv)

    degp = _make_deg_kernel(n_chunks, n_pad, rows_pt)(row_d, ones_c, z1_c)
    degp = degp.reshape(NC, n_pad)[:, :N].T

    g = pl.pallas_call(
        _tc2_body,
        grid=(grid_n,),
        in_specs=[
            pl.BlockSpec((BM, NC), lambda i: (i, 0)),
            pl.BlockSpec((BM, H), lambda i: (i, 0)),
        ],
        out_specs=pl.BlockSpec((BM, H), lambda i: (i, 0)),
        out_shape=jax.ShapeDtypeStruct((N, H), jnp.float32),
    )(degp, h)

    tmpp = _make_scatter_kernel(n_chunks, n_pad, rows_pt, H)(row_g, col_s, g, z2_c)
    tmpp = tmpp.reshape(NC, n_pad, H)[:, :N]

    y = pl.pallas_call(
        _tc3_body,
        grid=(grid_n,),
        in_specs=[
            pl.BlockSpec((NC, BM, H), lambda i: (0, i, 0)),
            pl.BlockSpec((BM, H), lambda i: (i, 0)),
            pl.BlockSpec((BM, NC), lambda i: (i, 0)),
            pl.BlockSpec((H, H), lambda i: (0, 0)),
            pl.BlockSpec((1, H), lambda i: (0, 0)),
            pl.BlockSpec((H, O), lambda i: (0, 0)),
            pl.BlockSpec((1, O), lambda i: (0, 0)),
        ],
        out_specs=pl.BlockSpec((BM, O), lambda i: (i, 0)),
        out_shape=jax.ShapeDtypeStruct((N, O), jnp.float32),
    )(tmpp, g, degp, W_h, b_h.reshape(1, H), W_o, b_o.reshape(1, O))
    return y


# speedup vs baseline: 2.4703x; 1.0458x over previous
import functools

import jax
import jax.numpy as jnp
from jax import lax
from jax.experimental import pallas as pl
from jax.experimental.pallas import tpu as pltpu
from jax.experimental.pallas import tpu_sc as plsc

NC = 2
NS = 16
NW = NC * NS
CHUNK = 128
NBUF = 4


def _tc1_body(x_ref, wi_ref, bi_ref, wc_ref, h_ref):
    a = jnp.dot(x_ref[...], wi_ref[...], preferred_element_type=jnp.float32)
    a = jnp.maximum(a + bi_ref[...], 0.0)
    h_ref[...] = jnp.dot(a, wc_ref[...], preferred_element_type=jnp.float32)


def _tc2_body(degp_ref, h_ref, g_ref):
    deg = degp_ref[:, 0] + degp_ref[:, 1] + 1.0
    dinv = lax.rsqrt(deg)
    g_ref[...] = h_ref[...] * dinv[:, None]


def _tc3_body(tmp_ref, g_ref, degp_ref, wh_ref, bh_ref, wo_ref, bo_ref, y_ref):
    deg = degp_ref[:, 0] + degp_ref[:, 1] + 1.0
    dinv = lax.rsqrt(deg)
    s = (tmp_ref[0] + tmp_ref[1] + g_ref[...]) * dinv[:, None]
    z = jnp.dot(s, wh_ref[...], preferred_element_type=jnp.float32)
    z = jnp.maximum(z + bh_ref[...], 0.0)
    y_ref[...] = jnp.dot(z, wo_ref[...], preferred_element_type=jnp.float32) + bo_ref[...]


def _make_deg_kernel(n_chunks, n_pad, rows_pt):
    mesh = plsc.VectorSubcoreMesh(core_axis_name="c", subcore_axis_name="s")

    @functools.partial(
        pl.kernel,
        mesh=mesh,
        out_type=jax.ShapeDtypeStruct((NC * n_pad,), jnp.float32),
        scratch_types=[
            pltpu.VMEM((n_chunks, CHUNK), jnp.int32),
            pltpu.VMEM((CHUNK,), jnp.float32),
            pltpu.VMEM_SHARED((n_pad,), jnp.float32),
            pltpu.SemaphoreType.DMA,
        ],
        compiler_params=pltpu.CompilerParams(use_tc_tiling_on_sc=False),
    )
    def deg_kernel(row_hbm, ones_hbm, z1_hbm, deg_out, idx_v, ones_v, acc, sem):
        cid = lax.axis_index("c")
        sid = lax.axis_index("s")
        wid = cid * NS + sid
        pltpu.sync_copy(row_hbm.at[wid], idx_v)
        pltpu.sync_copy(ones_hbm, ones_v)
        pltpu.sync_copy(z1_hbm, acc.at[pl.ds(sid * rows_pt, rows_pt)])
        plsc.subcore_barrier()

        def body(j, carry):
            pltpu.async_copy(ones_v, acc.at[idx_v.at[j]], sem, add=True)
            return carry

        lax.fori_loop(0, n_chunks, body, 0)

        def drain(j, carry):
            pltpu.make_async_copy(ones_v, acc.at[idx_v.at[j]], sem).wait()
            return carry

        lax.fori_loop(0, n_chunks, drain, 0)
        plsc.subcore_barrier()
        pltpu.sync_copy(acc.at[pl.ds(sid * rows_pt, rows_pt)],
                        deg_out.at[pl.ds(cid * n_pad + sid * rows_pt, rows_pt)])

    return deg_kernel


def _make_scatter_kernel(n_chunks, n_pad, rows_pt, h_dim):
    mesh = plsc.VectorSubcoreMesh(core_axis_name="c", subcore_axis_name="s")

    @functools.partial(
        pl.kernel,
        mesh=mesh,
        out_type=jax.ShapeDtypeStruct((NC, n_pad, h_dim), jnp.float32),
        scratch_types=[
            pltpu.VMEM((n_chunks, CHUNK), jnp.int32),
            pltpu.VMEM((n_chunks, CHUNK), jnp.int32),
            pltpu.VMEM((NBUF, CHUNK, h_dim), jnp.float32),
            pltpu.VMEM_SHARED((n_pad, h_dim), jnp.float32),
        ] + [pltpu.SemaphoreType.DMA] * (2 * NBUF),
        compiler_params=pltpu.CompilerParams(use_tc_tiling_on_sc=False),
    )
    def scatter_kernel(row_hbm, col_hbm, g_hbm, z2_hbm, tmp_out,
                       row_v, col_v, vbufs, acc, *sems):
        cid = lax.axis_index("c")
        sid = lax.axis_index("s")
        wid = cid * NS + sid
        pltpu.sync_copy(row_hbm.at[wid], row_v)
        pltpu.sync_copy(col_hbm.at[wid], col_v)
        pltpu.sync_copy(z2_hbm, acc.at[pl.ds(sid * rows_pt, rows_pt)])
        plsc.subcore_barrier()

        gsems = sems[:NBUF]
        ssems = sems[NBUF:]
        for b in range(NBUF):
            pltpu.async_copy(g_hbm.at[row_v.at[b]], vbufs.at[b], gsems[b])

        n_groups = n_chunks // NBUF

        def group(gi, carry):
            for b in range(NBUF):
                j = gi * NBUF + b
                pltpu.make_async_copy(
                    g_hbm.at[row_v.at[j]], vbufs.at[b], gsems[b]).wait()
                pltpu.async_copy(
                    vbufs.at[b], acc.at[col_v.at[j]], ssems[b], add=True)
            for b in range(NBUF):
                j = gi * NBUF + b
                pltpu.make_async_copy(
                    vbufs.at[b], acc.at[col_v.at[j]], ssems[b]).wait()

                @pl.when(gi < n_groups - 1)
                def _():
                    pltpu.async_copy(
                        g_hbm.at[row_v.at[j + NBUF]], vbufs.at[b], gsems[b])
            return carry

        lax.fori_loop(0, n_groups, group, 0)
        plsc.subcore_barrier()
        pltpu.sync_copy(acc.at[pl.ds(sid * rows_pt, rows_pt)],
                        tmp_out.at[cid, pl.ds(sid * rows_pt, rows_pt)])

    return scatter_kernel


def kernel(x, edge_index, W_i, b_i, W_conv, W_h, b_h, W_o, b_o):
    N, D = x.shape
    H = W_i.shape[1]
    O = W_o.shape[1]
    E = edge_index.shape[1]

    BM = 2000
    assert N % BM == 0
    grid_n = N // BM

    epw = -(-E // NW)
    n_chunks = -(-(-(-epw // CHUNK)) // NBUF) * NBUF
    e_pad = n_chunks * CHUNK * NW
    rows_pt = -(-N // (NS * 8)) * 8
    n_pad = rows_pt * NS

    row = edge_index[0]
    col = edge_index[1]
    pad = e_pad - E
    pad_ar = jnp.arange(pad, dtype=jnp.int32)
    dump_pad = N + pad_ar % (n_pad - N)
    row_d = jnp.concatenate([row, dump_pad]).reshape(NW, n_chunks, CHUNK)
    row_g = jnp.concatenate([row, pad_ar % N]).reshape(NW, n_chunks, CHUNK)
    col_s = jnp.concatenate([col, dump_pad]).reshape(NW, n_chunks, CHUNK)

    ones_c = jnp.ones((CHUNK,), jnp.float32)
    z1_c = jnp.zeros((rows_pt,), jnp.float32)
    z2_c = jnp.zeros((rows_pt, H), jnp.float32)

    h = pl.pallas_call(
        _tc1_body,
        grid=(grid_n,),
        in_specs=[
            pl.BlockSpec((BM, D), lambda i: (i, 0)),
            pl.BlockSpec((D, H), lambda i: (0, 0)),
            pl.BlockSpec((1, H), lambda i: (0, 0)),
            pl.BlockSpec((H, H), lambda i: (0, 0)),
        ],
        out_specs=pl.BlockSpec((BM, H), lambda i: (i, 0)),
        out_shape=jax.ShapeDtypeStruct((N, H), jnp.float32),
    )(x, W_i, b_i.reshape(1, H), W_conv)

    degp = _make_deg_kernel(n_chunks, n_pad, rows_pt)(row_d, ones_c, z1_c)
    degp = degp.reshape(NC, n_pad)[:, :N].T

    g = pl.pallas_call(
        _tc2_body,
        grid=(grid_n,),
        in_specs=[
            pl.BlockSpec((BM, NC), lambda i: (i, 0)),
            pl.BlockSpec((BM, H), lambda i: (i, 0)),
        ],
        out_specs=pl.BlockSpec((BM, H), lambda i: (i, 0)),
        out_shape=jax.ShapeDtypeStruct((N, H), jnp.float32),
    )(degp, h)

    tmpp = _make_scatter_kernel(n_chunks, n_pad, rows_pt, H)(row_g, col_s, g, z2_c)

    y = pl.pallas_call(
        _tc3_body,
        grid=(grid_n,),
        in_specs=[
            pl.BlockSpec((NC, BM, H), lambda i: (0, i, 0)),
            pl.BlockSpec((BM, H), lambda i: (i, 0)),
            pl.BlockSpec((BM, NC), lambda i: (i, 0)),
            pl.BlockSpec((H, H), lambda i: (0, 0)),
            pl.BlockSpec((1, H), lambda i: (0, 0)),
            pl.BlockSpec((H, O), lambda i: (0, 0)),
            pl.BlockSpec((1, O), lambda i: (0, 0)),
        ],
        out_specs=pl.BlockSpec((BM, O), lambda i: (i, 0)),
        out_shape=jax.ShapeDtypeStruct((N, O), jnp.float32),
    )(tmpp, g, degp, W_h, b_h.reshape(1, H), W_o, b_o.reshape(1, O))
    return y


# speedup vs baseline: 2.6328x; 1.0658x over previous
import functools

import jax
import jax.numpy as jnp
from jax import lax
from jax.experimental import pallas as pl
from jax.experimental.pallas import tpu as pltpu
from jax.experimental.pallas import tpu_sc as plsc

NC = 2
NS = 16
NW = NC * NS
CHUNK = 128
NBUF = 4


def _tc1_body(x_ref, wi_ref, bi_ref, wc_ref, h_ref):
    a = jnp.dot(x_ref[...], wi_ref[...], preferred_element_type=jnp.float32)
    a = jnp.maximum(a + bi_ref[...], 0.0)
    h_ref[...] = jnp.dot(a, wc_ref[...], preferred_element_type=jnp.float32)


def _tc2_body(degp_ref, h_ref, g_ref):
    deg = degp_ref[:, 0] + degp_ref[:, 1] + 1.0
    dinv = lax.rsqrt(deg)
    g_ref[...] = h_ref[...] * dinv[:, None]


def _tc3_body(tmp_ref, g_ref, degp_ref, wh_ref, bh_ref, wo_ref, bo_ref, y_ref):
    deg = degp_ref[:, 0] + degp_ref[:, 1] + 1.0
    dinv = lax.rsqrt(deg)
    s = (tmp_ref[0] + tmp_ref[1] + g_ref[...]) * dinv[:, None]
    z = jnp.dot(s, wh_ref[...], preferred_element_type=jnp.float32)
    z = jnp.maximum(z + bh_ref[...], 0.0)
    y_ref[...] = jnp.dot(z, wo_ref[...], preferred_element_type=jnp.float32) + bo_ref[...]


def _make_deg_kernel(n_chunks, n_pad, rows_pt):
    mesh = plsc.VectorSubcoreMesh(core_axis_name="c", subcore_axis_name="s")

    @functools.partial(
        pl.kernel,
        mesh=mesh,
        out_type=jax.ShapeDtypeStruct((NC * n_pad,), jnp.float32),
        scratch_types=[
            pltpu.VMEM((n_chunks, CHUNK), jnp.int32),
            pltpu.VMEM((CHUNK,), jnp.float32),
            pltpu.VMEM_SHARED((n_pad,), jnp.float32),
            pltpu.SemaphoreType.DMA,
        ],
        compiler_params=pltpu.CompilerParams(use_tc_tiling_on_sc=False),
    )
    def deg_kernel(row_hbm, ones_hbm, z1_hbm, deg_out, idx_v, ones_v, acc, sem):
        cid = lax.axis_index("c")
        sid = lax.axis_index("s")
        wid = cid * NS + sid
        pltpu.sync_copy(row_hbm.at[wid], idx_v)
        pltpu.sync_copy(ones_hbm, ones_v)
        pltpu.sync_copy(z1_hbm, acc.at[pl.ds(sid * rows_pt, rows_pt)])
        plsc.subcore_barrier()

        def body(j, carry):
            pltpu.async_copy(ones_v, acc.at[idx_v.at[j]], sem, add=True)
            return carry

        lax.fori_loop(0, n_chunks, body, 0)

        def drain(j, carry):
            pltpu.make_async_copy(ones_v, acc.at[idx_v.at[j]], sem).wait()
            return carry

        lax.fori_loop(0, n_chunks, drain, 0)
        plsc.subcore_barrier()
        pltpu.sync_copy(acc.at[pl.ds(sid * rows_pt, rows_pt)],
                        deg_out.at[pl.ds(cid * n_pad + sid * rows_pt, rows_pt)])

    return deg_kernel


def _make_scatter_kernel(n_chunks, n_pad, rows_pt, h_dim):
    mesh = plsc.VectorSubcoreMesh(core_axis_name="c", subcore_axis_name="s")

    @functools.partial(
        pl.kernel,
        mesh=mesh,
        out_type=jax.ShapeDtypeStruct((NC, n_pad, h_dim), jnp.float32),
        scratch_types=[
            pltpu.VMEM((n_chunks, CHUNK), jnp.int32),
            pltpu.VMEM((n_chunks, CHUNK), jnp.int32),
            pltpu.VMEM((NBUF, CHUNK, h_dim), jnp.float32),
            pltpu.VMEM_SHARED((n_pad, h_dim), jnp.float32),
        ] + [pltpu.SemaphoreType.DMA] * NBUF,
        compiler_params=pltpu.CompilerParams(use_tc_tiling_on_sc=False),
    )
    def scatter_kernel(row_hbm, col_hbm, g_hbm, z2_hbm, tmp_out,
                       row_v, col_v, vbufs, acc, *sems):
        cid = lax.axis_index("c")
        sid = lax.axis_index("s")
        wid = cid * NS + sid
        pltpu.sync_copy(row_hbm.at[wid], row_v)
        pltpu.sync_copy(col_hbm.at[wid], col_v)
        pltpu.sync_copy(z2_hbm, acc.at[pl.ds(sid * rows_pt, rows_pt)])
        plsc.subcore_barrier()

        gsems = sems[:NBUF]
        for b in range(NBUF):
            pltpu.async_copy(g_hbm.at[row_v.at[b]], vbufs.at[b], gsems[b])

        n_groups = n_chunks // NBUF

        def group(gi, carry):
            for b in range(NBUF):
                j = gi * NBUF + b
                pltpu.make_async_copy(
                    g_hbm.at[row_v.at[j]], vbufs.at[b], gsems[b]).wait()
                pltpu.sync_copy(vbufs.at[b], acc.at[col_v.at[j]], add=True)

                @pl.when(gi < n_groups - 1)
                def _():
                    pltpu.async_copy(
                        g_hbm.at[row_v.at[j + NBUF]], vbufs.at[b], gsems[b])
            return carry

        lax.fori_loop(0, n_groups, group, 0)
        plsc.subcore_barrier()
        pltpu.sync_copy(acc.at[pl.ds(sid * rows_pt, rows_pt)],
                        tmp_out.at[cid, pl.ds(sid * rows_pt, rows_pt)])

    return scatter_kernel


def kernel(x, edge_index, W_i, b_i, W_conv, W_h, b_h, W_o, b_o):
    N, D = x.shape
    H = W_i.shape[1]
    O = W_o.shape[1]
    E = edge_index.shape[1]

    BM = 2000
    assert N % BM == 0
    grid_n = N // BM

    epw = -(-E // NW)
    n_chunks = -(-(-(-epw // CHUNK)) // NBUF) * NBUF
    e_pad = n_chunks * CHUNK * NW
    rows_pt = -(-N // (NS * 8)) * 8
    n_pad = rows_pt * NS

    row = edge_index[0]
    col = edge_index[1]
    pad = e_pad - E
    pad_ar = jnp.arange(pad, dtype=jnp.int32)
    dump_pad = N + pad_ar % (n_pad - N)
    row_d = jnp.concatenate([row, dump_pad]).reshape(NW, n_chunks, CHUNK)
    row_g = jnp.concatenate([row, pad_ar % N]).reshape(NW, n_chunks, CHUNK)
    col_s = jnp.concatenate([col, dump_pad]).reshape(NW, n_chunks, CHUNK)

    ones_c = jnp.ones((CHUNK,), jnp.float32)
    z1_c = jnp.zeros((rows_pt,), jnp.float32)
    z2_c = jnp.zeros((rows_pt, H), jnp.float32)

    h = pl.pallas_call(
        _tc1_body,
        grid=(grid_n,),
        in_specs=[
            pl.BlockSpec((BM, D), lambda i: (i, 0)),
            pl.BlockSpec((D, H), lambda i: (0, 0)),
            pl.BlockSpec((1, H), lambda i: (0, 0)),
            pl.BlockSpec((H, H), lambda i: (0, 0)),
        ],
        out_specs=pl.BlockSpec((BM, H), lambda i: (i, 0)),
        out_shape=jax.ShapeDtypeStruct((N, H), jnp.float32),
    )(x, W_i, b_i.reshape(1, H), W_conv)

    degp = _make_deg_kernel(n_chunks, n_pad, rows_pt)(row_d, ones_c, z1_c)
    degp = degp.reshape(NC, n_pad)[:, :N].T

    g = pl.pallas_call(
        _tc2_body,
        grid=(grid_n,),
        in_specs=[
            pl.BlockSpec((BM, NC), lambda i: (i, 0)),
            pl.BlockSpec((BM, H), lambda i: (i, 0)),
        ],
        out_specs=pl.BlockSpec((BM, H), lambda i: (i, 0)),
        out_shape=jax.ShapeDtypeStruct((N, H), jnp.float32),
    )(degp, h)

    tmpp = _make_scatter_kernel(n_chunks, n_pad, rows_pt, H)(row_g, col_s, g, z2_c)

    y = pl.pallas_call(
        _tc3_body,
        grid=(grid_n,),
        in_specs=[
            pl.BlockSpec((NC, BM, H), lambda i: (0, i, 0)),
            pl.BlockSpec((BM, H), lambda i: (i, 0)),
            pl.BlockSpec((BM, NC), lambda i: (i, 0)),
            pl.BlockSpec((H, H), lambda i: (0, 0)),
            pl.BlockSpec((1, H), lambda i: (0, 0)),
            pl.BlockSpec((H, O), lambda i: (0, 0)),
            pl.BlockSpec((1, O), lambda i: (0, 0)),
        ],
        out_specs=pl.BlockSpec((BM, O), lambda i: (i, 0)),
        out_shape=jax.ShapeDtypeStruct((N, O), jnp.float32),
    )(tmpp, g, degp, W_h, b_h.reshape(1, H), W_o, b_o.reshape(1, O))
    return y
